# baseline (speedup 1.0000x reference)
import jax
import jax.numpy as jnp
from jax.experimental import pallas as pl
from jax.experimental.pallas import tpu as pltpu


def _add_body(x_ref, emb_ref, out_ref):
    out_ref[...] = x_ref[...] + emb_ref[None]


def kernel(x, emb):
    batch, seq_len, d_model = x.shape
    blk = 512
    n_seq = seq_len // blk
    return pl.pallas_call(
        _add_body,
        grid=(n_seq,),
        in_specs=[
            pl.BlockSpec((batch, blk, d_model), lambda s: (0, s, 0)),
            pl.BlockSpec((blk, d_model), lambda s: (s, 0)),
        ],
        out_specs=pl.BlockSpec((batch, blk, d_model), lambda s: (0, s, 0)),
        out_shape=jax.ShapeDtypeStruct((batch, seq_len, d_model), x.dtype),
    )(x, emb)


# speedup vs baseline: 1.0079x; 1.0079x over previous
import jax
import jax.numpy as jnp
from jax.experimental import pallas as pl
from jax.experimental.pallas import tpu as pltpu


def _add_body(x_ref, emb_ref, out_ref):
    out_ref[...] = x_ref[...] + emb_ref[None]


def kernel(x, emb):
    batch, seq_len, d_model = x.shape
    blk = 1024
    n_seq = seq_len // blk
    return pl.pallas_call(
        _add_body,
        grid=(n_seq, 2),
        in_specs=[
            pl.BlockSpec((2, blk, d_model), lambda s, b: (b, s, 0)),
            pl.BlockSpec((blk, d_model), lambda s, b: (s, 0)),
        ],
        out_specs=pl.BlockSpec((2, blk, d_model), lambda s, b: (b, s, 0)),
        out_shape=jax.ShapeDtypeStruct((batch, seq_len, d_model), x.dtype),
    )(x, emb)
